# baseline (speedup 1.0000x reference)
import functools

import jax
import jax.numpy as jnp
import numpy as np
from jax import lax
from jax.experimental import pallas as pl
from jax.experimental.pallas import tpu as pltpu
from jax.experimental.pallas import tpu_sc as plsc

N_NODES = 10000
N_EDGES = 160000
IN_CH = 256
HID = 512
NUM_TEAMS = 30

N_PAD = 10240
E_ROWS = 1280
E_PAD = E_ROWS * 128
PAD_NODE = 10200
STRIPE = N_PAD // 16
NB = 512
GRID = N_PAD // NB

_SC_MESH = dict(core_axis_name="c", subcore_axis_name="s")


def _league_mat() -> np.ndarray:
    iu = np.triu_indices(NUM_TEAMS, k=1)
    lsrc, ldst = iu[0], iu[1]
    deg = np.zeros((NUM_TEAMS,), np.float64)
    np.add.at(deg, ldst, 1.0)
    deg += 1.0
    dis = 1.0 / np.sqrt(deg)
    L = np.zeros((32, 32), np.float64)
    L[ldst, lsrc] = dis[lsrc] * dis[ldst]
    L[np.arange(NUM_TEAMS), np.arange(NUM_TEAMS)] = dis * dis
    return L.astype(np.float32)


_LEAGUE_L = _league_mat()


def _sc_counts_body(dst_hbm, ones_hbm, zeros_hbm, out_hbm, dstv, onesv, acc):
    c = lax.axis_index("c")
    s = lax.axis_index("s")
    w = s * 2 + c
    pltpu.sync_copy(ones_hbm, onesv)
    for z in range(5):
        pltpu.sync_copy(zeros_hbm, acc.at[pl.ds(s * STRIPE + z * 128, 128)])
    pltpu.sync_copy(dst_hbm.at[pl.ds(w * 40, 40)], dstv)
    plsc.subcore_barrier()

    def chunk(j, carry):
        pltpu.sync_copy(onesv, acc.at[dstv.at[j]], add=True)
        return carry

    lax.fori_loop(0, 40, chunk, 0)
    plsc.subcore_barrier()
    pltpu.sync_copy(acc.at[pl.ds(s * STRIPE, STRIPE)],
                    out_hbm.at[c, pl.ds(s * STRIPE, STRIPE)])


_sc_counts = pl.kernel(
    _sc_counts_body,
    out_type=jax.ShapeDtypeStruct((2, N_PAD, 128), jnp.float32),
    mesh=plsc.VectorSubcoreMesh(**_SC_MESH),
    scratch_types=[
        pltpu.VMEM((40, 128), jnp.int32),
        pltpu.VMEM((128, 128), jnp.float32),
        pltpu.VMEM_SHARED((N_PAD, 128), jnp.float32),
    ],
)


def _make_agg(n_groups):
    gpc = n_groups // 2

    def body(xs_hbm, src_hbm, dst_hbm, zeros_hbm, out_hbm,
             srcv, dstv, b0, b1, b2, b3, acc, g0, g1, g2, g3):
        c = lax.axis_index("c")
        s = lax.axis_index("s")
        bufs = (b0, b1, b2, b3)
        gsem = (g0, g1, g2, g3)
        pltpu.sync_copy(dst_hbm.at[pl.ds(s * 80, 80)], dstv)
        for it in range(gpc):
            g = it * 2 + c
            pltpu.sync_copy(src_hbm.at[g, pl.ds(s * 80, 80)], srcv)
            for z in range(5):
                pltpu.sync_copy(zeros_hbm,
                                acc.at[pl.ds(s * STRIPE + z * 128, 128)])
            plsc.subcore_barrier()

            def chunk(j, carry):
                pltpu.sync_copy(xs_hbm.at[srcv.at[j]], bufs[0])
                pltpu.sync_copy(bufs[0], acc.at[dstv.at[j]], add=True)
                return carry

            lax.fori_loop(0, 80, chunk, 0)
            plsc.subcore_barrier()
            pltpu.sync_copy(acc.at[pl.ds(s * STRIPE, STRIPE)],
                            out_hbm.at[g, pl.ds(s * STRIPE, STRIPE)])

    return pl.kernel(
        body,
        out_type=jax.ShapeDtypeStruct((n_groups, N_PAD, 128), jnp.float32),
        mesh=plsc.VectorSubcoreMesh(**_SC_MESH),
        scratch_types=[
            pltpu.VMEM((80, 128), jnp.int32),
            pltpu.VMEM((80, 128), jnp.int32),
            pltpu.VMEM((128, 128), jnp.float32),
            pltpu.VMEM((128, 128), jnp.float32),
            pltpu.VMEM((128, 128), jnp.float32),
            pltpu.VMEM((128, 128), jnp.float32),
            pltpu.VMEM_SHARED((N_PAD, 128), jnp.float32),
            pltpu.SemaphoreType.DMA,
            pltpu.SemaphoreType.DMA,
            pltpu.SemaphoreType.DMA,
            pltpu.SemaphoreType.DMA,
        ],
    )


_sc_agg2 = _make_agg(2)
_sc_agg4 = _make_agg(4)


def _dis_from_counts(cnt_ref):
    deg = cnt_ref[0, :, 0:1] + cnt_ref[1, :, 0:1] + 1.0
    dis = lax.rsqrt(deg)
    return dis


def _ln_lrelu(h, w, b, eps=1e-5):
    m = jnp.mean(h, axis=-1, keepdims=True)
    v = jnp.mean((h - m) ** 2, axis=-1, keepdims=True)
    h = (h - m) * lax.rsqrt(v + eps) * w + b
    return jnp.where(h >= 0, h, 0.01 * h)


def _tc_a_body(x_ref, cnt_ref, o_ref):
    dis = _dis_from_counts(cnt_ref)
    for g in range(2):
        o_ref[g] = dis * x_ref[:, g * 128:(g + 1) * 128]


_tc_a = pl.pallas_call(
    _tc_a_body,
    grid=(GRID,),
    in_specs=[pl.BlockSpec((NB, IN_CH), lambda i: (i, 0)),
              pl.BlockSpec((2, NB, 128), lambda i: (0, i, 0))],
    out_specs=pl.BlockSpec((2, NB, 128), lambda i: (0, i, 0)),
    out_shape=jax.ShapeDtypeStruct((2, N_PAD, 128), jnp.float32),
)


def _tc_m2_body(x_ref, w1_ref, o_ref):
    o_ref[...] = jnp.dot(x_ref[...], w1_ref[...],
                         preferred_element_type=jnp.float32)


_tc_m2 = pl.pallas_call(
    _tc_m2_body,
    grid=(GRID,),
    in_specs=[pl.BlockSpec((NB, IN_CH), lambda i: (i, 0)),
              pl.BlockSpec((IN_CH, HID), lambda i: (0, 0))],
    out_specs=pl.BlockSpec((NB, HID), lambda i: (i, 0)),
    out_shape=jax.ShapeDtypeStruct((N_PAD, HID), jnp.float32),
)


def _tc_b_body(s1_ref, m2_ref, cnt_ref, w1_ref, b1_ref, lw_ref, lb_ref,
               w2_ref, go_ref, gs_ref):
    dis = _dis_from_counts(cnt_ref)
    dis2 = dis * dis
    m1 = (jnp.dot(s1_ref[0], w1_ref[0:128, :],
                  preferred_element_type=jnp.float32) +
          jnp.dot(s1_ref[1], w1_ref[128:256, :],
                  preferred_element_type=jnp.float32))
    pre = dis * m1 + dis2 * m2_ref[...] + b1_ref[...]
    h = _ln_lrelu(pre, lw_ref[...], lb_ref[...])
    gmat = jnp.dot(h, w2_ref[...], preferred_element_type=jnp.float32)
    go_ref[...] = gmat
    for g in range(4):
        gs_ref[g] = dis * gmat[:, g * 128:(g + 1) * 128]


_tc_b = pl.pallas_call(
    _tc_b_body,
    grid=(GRID,),
    in_specs=[pl.BlockSpec((2, NB, 128), lambda i: (0, i, 0)),
              pl.BlockSpec((NB, HID), lambda i: (i, 0)),
              pl.BlockSpec((2, NB, 128), lambda i: (0, i, 0)),
              pl.BlockSpec((IN_CH, HID), lambda i: (0, 0)),
              pl.BlockSpec((1, HID), lambda i: (0, 0)),
              pl.BlockSpec((1, HID), lambda i: (0, 0)),
              pl.BlockSpec((1, HID), lambda i: (0, 0)),
              pl.BlockSpec((HID, HID), lambda i: (0, 0))],
    out_specs=[pl.BlockSpec((NB, HID), lambda i: (i, 0)),
               pl.BlockSpec((4, NB, 128), lambda i: (0, i, 0))],
    out_shape=[jax.ShapeDtypeStruct((N_PAD, HID), jnp.float32),
               jax.ShapeDtypeStruct((4, N_PAD, 128), jnp.float32)],
)


def _tc_c_body(s2_ref, g_ref, cnt_ref, b2_ref, lw_ref, lb_ref, batch_ref,
               sums_ref, cnto_ref):
    i = pl.program_id(0)
    dis = _dis_from_counts(cnt_ref)
    dis2 = dis * dis
    cat = jnp.concatenate([s2_ref[0], s2_ref[1], s2_ref[2], s2_ref[3]],
                          axis=1)
    pre = dis * cat + dis2 * g_ref[...] + b2_ref[...]
    h2 = _ln_lrelu(pre, lw_ref[...], lb_ref[...])
    ti = lax.broadcasted_iota(jnp.int32, (32, NB), 0)
    P = (batch_ref[...] == ti).astype(jnp.float32)
    su = jnp.dot(P, h2, preferred_element_type=jnp.float32)
    cn = jnp.broadcast_to(jnp.sum(P, axis=1, keepdims=True), (32, 128))

    @pl.when(i == 0)
    def _():
        sums_ref[...] = jnp.zeros_like(sums_ref)
        cnto_ref[...] = jnp.zeros_like(cnto_ref)

    sums_ref[...] += su
    cnto_ref[...] += cn


_tc_c = pl.pallas_call(
    _tc_c_body,
    grid=(GRID,),
    in_specs=[pl.BlockSpec((4, NB, 128), lambda i: (0, i, 0)),
              pl.BlockSpec((NB, HID), lambda i: (i, 0)),
              pl.BlockSpec((2, NB, 128), lambda i: (0, i, 0)),
              pl.BlockSpec((1, HID), lambda i: (0, 0)),
              pl.BlockSpec((1, HID), lambda i: (0, 0)),
              pl.BlockSpec((1, HID), lambda i: (0, 0)),
              pl.BlockSpec((1, NB), lambda i: (0, i))],
    out_specs=[pl.BlockSpec((32, HID), lambda i: (0, 0)),
               pl.BlockSpec((32, 128), lambda i: (0, 0))],
    out_shape=[jax.ShapeDtypeStruct((32, HID), jnp.float32),
               jax.ShapeDtypeStruct((32, 128), jnp.float32)],
)


def _tc_d_body(sums_ref, cnt_ref, L_ref, wl1_ref, bl1_ref, l1w_ref, l1b_ref,
               wl2_ref, bl2_ref, l2w_ref, l2b_ref, fcw_ref, fcb_ref, o_ref):
    cnt = jnp.maximum(cnt_ref[:, 0:1], 1.0)
    team = sums_ref[...] / cnt
    L = L_ref[...]
    t = jnp.dot(L, jnp.dot(team, wl1_ref[...],
                           preferred_element_type=jnp.float32),
                preferred_element_type=jnp.float32) + bl1_ref[...]
    t = _ln_lrelu(t, l1w_ref[...], l1b_ref[...])
    t = jnp.dot(L, jnp.dot(t, wl2_ref[...],
                           preferred_element_type=jnp.float32),
                preferred_element_type=jnp.float32) + bl2_ref[...]
    t = _ln_lrelu(t, l2w_ref[...], l2b_ref[...])
    t = t + team
    o_ref[...] = jnp.dot(t, fcw_ref[...],
                         preferred_element_type=jnp.float32) + fcb_ref[...]


_tc_d = pl.pallas_call(
    _tc_d_body,
    out_shape=jax.ShapeDtypeStruct((32, 128), jnp.float32),
)


def kernel(x, edge_index, batch, W1, b1, ln1_w, ln1_b, W2, b2, ln2_w, ln2_b,
           Wl1, bl1, lnl1_w, lnl1_b, Wl2, bl2, lnl2_w, lnl2_b, fc_w, fc_b):
    src, dst = edge_index[0], edge_index[1]
    pad = PAD_NODE + (jnp.arange(E_PAD - N_EDGES, dtype=jnp.int32) % 40)
    srcp = jnp.concatenate([src, pad]).reshape(E_ROWS, 128)
    dstp = jnp.concatenate([dst, pad]).reshape(E_ROWS, 128)
    off2 = (jnp.arange(2, dtype=jnp.int32) * N_PAD)[:, None, None]
    off4 = (jnp.arange(4, dtype=jnp.int32) * N_PAD)[:, None, None]
    src2 = srcp[None] + off2
    src4 = srcp[None] + off4
    xp = jnp.concatenate(
        [x, jnp.zeros((N_PAD - N_NODES, IN_CH), jnp.float32)])
    batch2d = jnp.concatenate(
        [batch, jnp.full((N_PAD - N_NODES,), 99, jnp.int32)]).reshape(1, N_PAD)
    zeros128 = jnp.zeros((128, 128), jnp.float32)
    ones128 = jnp.ones((128, 128), jnp.float32)

    m2 = _tc_m2(xp, W1)
    cnts = _sc_counts(dstp, ones128, zeros128)
    xs2 = _tc_a(xp, cnts)
    s1 = _sc_agg2(xs2.reshape(2 * N_PAD, 128), src2, dstp, zeros128)
    gmat, gs4 = _tc_b(s1, m2, cnts, W1, b1.reshape(1, HID),
                      ln1_w.reshape(1, HID), ln1_b.reshape(1, HID), W2)
    s2 = _sc_agg4(gs4.reshape(4 * N_PAD, 128), src4, dstp, zeros128)
    sums, cnt = _tc_c(s2, gmat, cnts, b2.reshape(1, HID),
                      ln2_w.reshape(1, HID), ln2_b.reshape(1, HID), batch2d)
    outd = _tc_d(sums, cnt, jnp.asarray(_LEAGUE_L), Wl1,
                 bl1.reshape(1, HID), lnl1_w.reshape(1, HID),
                 lnl1_b.reshape(1, HID), Wl2, bl2.reshape(1, HID),
                 lnl2_w.reshape(1, HID), lnl2_b.reshape(1, HID),
                 jnp.broadcast_to(fc_w, (HID, 128)),
                 jnp.broadcast_to(fc_b.reshape(1, 1), (1, 128)))
    return outd[:NUM_TEAMS, 0]


# speedup vs baseline: 1.0123x; 1.0123x over previous
import functools

import jax
import jax.numpy as jnp
import numpy as np
from jax import lax
from jax.experimental import pallas as pl
from jax.experimental.pallas import tpu as pltpu
from jax.experimental.pallas import tpu_sc as plsc

N_NODES = 10000
N_EDGES = 160000
IN_CH = 256
HID = 512
NUM_TEAMS = 30

N_PAD = 10240
E_ROWS = 1280
E_PAD = E_ROWS * 128
PAD_NODE = 10200
STRIPE = N_PAD // 16
NB = 512
GRID = N_PAD // NB

_SC_MESH = dict(core_axis_name="c", subcore_axis_name="s")


def _league_mat() -> np.ndarray:
    iu = np.triu_indices(NUM_TEAMS, k=1)
    lsrc, ldst = iu[0], iu[1]
    deg = np.zeros((NUM_TEAMS,), np.float64)
    np.add.at(deg, ldst, 1.0)
    deg += 1.0
    dis = 1.0 / np.sqrt(deg)
    L = np.zeros((32, 32), np.float64)
    L[ldst, lsrc] = dis[lsrc] * dis[ldst]
    L[np.arange(NUM_TEAMS), np.arange(NUM_TEAMS)] = dis * dis
    return L.astype(np.float32)


_LEAGUE_L = _league_mat()


def _sc_counts_body(dst_hbm, ones_hbm, zeros_hbm, out_hbm, dstv, onesv, acc):
    c = lax.axis_index("c")
    s = lax.axis_index("s")
    w = s * 2 + c
    pltpu.sync_copy(ones_hbm, onesv)
    for z in range(5):
        pltpu.sync_copy(zeros_hbm, acc.at[pl.ds(s * STRIPE + z * 128, 128)])
    pltpu.sync_copy(dst_hbm.at[pl.ds(w * 40, 40)], dstv)
    plsc.subcore_barrier()

    def chunk(j, carry):
        pltpu.sync_copy(onesv, acc.at[dstv.at[j]], add=True)
        return carry

    lax.fori_loop(0, 40, chunk, 0)
    plsc.subcore_barrier()
    pltpu.sync_copy(acc.at[pl.ds(s * STRIPE, STRIPE)],
                    out_hbm.at[c, pl.ds(s * STRIPE, STRIPE)])


_sc_counts = pl.kernel(
    _sc_counts_body,
    out_type=jax.ShapeDtypeStruct((2, N_PAD, 128), jnp.float32),
    mesh=plsc.VectorSubcoreMesh(**_SC_MESH),
    scratch_types=[
        pltpu.VMEM((40, 128), jnp.int32),
        pltpu.VMEM((128, 128), jnp.float32),
        pltpu.VMEM_SHARED((N_PAD, 128), jnp.float32),
    ],
)


def _make_agg(n_groups):
    gpc = n_groups // 2

    def body(xs_hbm, src_hbm, dst_hbm, zeros_hbm, out_hbm,
             srcv, dstv, b0, b1, b2, b3, acc, g0, g1, g2, g3):
        c = lax.axis_index("c")
        s = lax.axis_index("s")
        bufs = (b0, b1, b2, b3)
        gsem = (g0, g1, g2, g3)
        pltpu.sync_copy(dst_hbm.at[pl.ds(s * 80, 80)], dstv)
        for it in range(gpc):
            g = it * 2 + c
            pltpu.sync_copy(src_hbm.at[g, pl.ds(s * 80, 80)], srcv)
            for z in range(5):
                pltpu.sync_copy(zeros_hbm,
                                acc.at[pl.ds(s * STRIPE + z * 128, 128)])
            plsc.subcore_barrier()

            def chunk(j, carry):
                pltpu.sync_copy(xs_hbm.at[srcv.at[j]], bufs[0])
                pltpu.sync_copy(bufs[0], acc.at[dstv.at[j]], add=True)
                return carry

            lax.fori_loop(0, 80, chunk, 0)
            plsc.subcore_barrier()
            pltpu.sync_copy(acc.at[pl.ds(s * STRIPE, STRIPE)],
                            out_hbm.at[g, pl.ds(s * STRIPE, STRIPE)])

    return pl.kernel(
        body,
        out_type=jax.ShapeDtypeStruct((n_groups, N_PAD, 128), jnp.float32),
        mesh=plsc.VectorSubcoreMesh(**_SC_MESH),
        scratch_types=[
            pltpu.VMEM((80, 128), jnp.int32),
            pltpu.VMEM((80, 128), jnp.int32),
            pltpu.VMEM((128, 128), jnp.float32),
            pltpu.VMEM((128, 128), jnp.float32),
            pltpu.VMEM((128, 128), jnp.float32),
            pltpu.VMEM((128, 128), jnp.float32),
            pltpu.VMEM_SHARED((N_PAD, 128), jnp.float32),
            pltpu.SemaphoreType.DMA,
            pltpu.SemaphoreType.DMA,
            pltpu.SemaphoreType.DMA,
            pltpu.SemaphoreType.DMA,
        ],
    )


_sc_agg2 = _make_agg(2)
_sc_agg4 = _make_agg(4)


def _dis_from_counts(cnt_ref):
    deg = cnt_ref[0, :, 0:1] + cnt_ref[1, :, 0:1] + 1.0
    dis = lax.rsqrt(deg)
    return dis


def _ln_lrelu(h, w, b, eps=1e-5):
    m = jnp.mean(h, axis=-1, keepdims=True)
    v = jnp.mean((h - m) ** 2, axis=-1, keepdims=True)
    h = (h - m) * lax.rsqrt(v + eps) * w + b
    return jnp.where(h >= 0, h, 0.01 * h)


def _tc_a_body(x_ref, cnt_ref, o_ref):
    dis = _dis_from_counts(cnt_ref)
    for g in range(2):
        o_ref[g] = dis * x_ref[:, g * 128:(g + 1) * 128]


_tc_a = pl.pallas_call(
    _tc_a_body,
    grid=(GRID,),
    in_specs=[pl.BlockSpec((NB, IN_CH), lambda i: (i, 0)),
              pl.BlockSpec((2, NB, 128), lambda i: (0, i, 0))],
    out_specs=pl.BlockSpec((2, NB, 128), lambda i: (0, i, 0)),
    out_shape=jax.ShapeDtypeStruct((2, N_PAD, 128), jnp.float32),
)


def _tc_b_body(s1_ref, x_ref, cnt_ref, w1_ref, b1_ref, lw_ref, lb_ref,
               w2_ref, go_ref, gs_ref):
    dis = _dis_from_counts(cnt_ref)
    dis2 = dis * dis
    m1 = (jnp.dot(s1_ref[0], w1_ref[0:128, :],
                  preferred_element_type=jnp.float32) +
          jnp.dot(s1_ref[1], w1_ref[128:256, :],
                  preferred_element_type=jnp.float32))
    m2 = jnp.dot(x_ref[...], w1_ref[...], preferred_element_type=jnp.float32)
    pre = dis * m1 + dis2 * m2 + b1_ref[...]
    h = _ln_lrelu(pre, lw_ref[...], lb_ref[...])
    gmat = jnp.dot(h, w2_ref[...], preferred_element_type=jnp.float32)
    go_ref[...] = gmat
    for g in range(4):
        gs_ref[g] = dis * gmat[:, g * 128:(g + 1) * 128]


_tc_b = pl.pallas_call(
    _tc_b_body,
    grid=(GRID,),
    in_specs=[pl.BlockSpec((2, NB, 128), lambda i: (0, i, 0)),
              pl.BlockSpec((NB, IN_CH), lambda i: (i, 0)),
              pl.BlockSpec((2, NB, 128), lambda i: (0, i, 0)),
              pl.BlockSpec((IN_CH, HID), lambda i: (0, 0)),
              pl.BlockSpec((1, HID), lambda i: (0, 0)),
              pl.BlockSpec((1, HID), lambda i: (0, 0)),
              pl.BlockSpec((1, HID), lambda i: (0, 0)),
              pl.BlockSpec((HID, HID), lambda i: (0, 0))],
    out_specs=[pl.BlockSpec((NB, HID), lambda i: (i, 0)),
               pl.BlockSpec((4, NB, 128), lambda i: (0, i, 0))],
    out_shape=[jax.ShapeDtypeStruct((N_PAD, HID), jnp.float32),
               jax.ShapeDtypeStruct((4, N_PAD, 128), jnp.float32)],
)


def _tc_c_body(s2_ref, g_ref, cnt_ref, b2_ref, lw_ref, lb_ref, batch_ref,
               L_ref, wl1_ref, bl1_ref, l1w_ref, l1b_ref,
               wl2_ref, bl2_ref, l2w_ref, l2b_ref, fcw_ref, fcb_ref,
               sums_ref, cnto_ref, po_ref):
    i = pl.program_id(0)
    dis = _dis_from_counts(cnt_ref)
    dis2 = dis * dis
    cat = jnp.concatenate([s2_ref[0], s2_ref[1], s2_ref[2], s2_ref[3]],
                          axis=1)
    pre = dis * cat + dis2 * g_ref[...] + b2_ref[...]
    h2 = _ln_lrelu(pre, lw_ref[...], lb_ref[...])
    ti = lax.broadcasted_iota(jnp.int32, (32, NB), 0)
    P = (batch_ref[...] == ti).astype(jnp.float32)
    su = jnp.dot(P, h2, preferred_element_type=jnp.float32)
    cn = jnp.broadcast_to(jnp.sum(P, axis=1, keepdims=True), (32, 128))

    @pl.when(i == 0)
    def _():
        sums_ref[...] = jnp.zeros_like(sums_ref)
        cnto_ref[...] = jnp.zeros_like(cnto_ref)

    sums_ref[...] += su
    cnto_ref[...] += cn

    @pl.when(i == GRID - 1)
    def _():
        cnt = jnp.maximum(cnto_ref[:, 0:1], 1.0)
        team = sums_ref[...] / cnt
        L = L_ref[...]
        tt = jnp.dot(L, jnp.dot(team, wl1_ref[...],
                                preferred_element_type=jnp.float32),
                     preferred_element_type=jnp.float32) + bl1_ref[...]
        tt = _ln_lrelu(tt, l1w_ref[...], l1b_ref[...])
        tt = jnp.dot(L, jnp.dot(tt, wl2_ref[...],
                                preferred_element_type=jnp.float32),
                     preferred_element_type=jnp.float32) + bl2_ref[...]
        tt = _ln_lrelu(tt, l2w_ref[...], l2b_ref[...])
        tt = tt + team
        po_ref[...] = jnp.dot(tt, fcw_ref[...],
                              preferred_element_type=jnp.float32) + fcb_ref[...]


_tc_c = pl.pallas_call(
    _tc_c_body,
    grid=(GRID,),
    in_specs=[pl.BlockSpec((4, NB, 128), lambda i: (0, i, 0)),
              pl.BlockSpec((NB, HID), lambda i: (i, 0)),
              pl.BlockSpec((2, NB, 128), lambda i: (0, i, 0)),
              pl.BlockSpec((1, HID), lambda i: (0, 0)),
              pl.BlockSpec((1, HID), lambda i: (0, 0)),
              pl.BlockSpec((1, HID), lambda i: (0, 0)),
              pl.BlockSpec((1, NB), lambda i: (0, i)),
              pl.BlockSpec((32, 32), lambda i: (0, 0)),
              pl.BlockSpec((HID, HID), lambda i: (0, 0)),
              pl.BlockSpec((1, HID), lambda i: (0, 0)),
              pl.BlockSpec((1, HID), lambda i: (0, 0)),
              pl.BlockSpec((1, HID), lambda i: (0, 0)),
              pl.BlockSpec((HID, HID), lambda i: (0, 0)),
              pl.BlockSpec((1, HID), lambda i: (0, 0)),
              pl.BlockSpec((1, HID), lambda i: (0, 0)),
              pl.BlockSpec((1, HID), lambda i: (0, 0)),
              pl.BlockSpec((HID, 128), lambda i: (0, 0)),
              pl.BlockSpec((1, 128), lambda i: (0, 0))],
    out_specs=[pl.BlockSpec((32, HID), lambda i: (0, 0)),
               pl.BlockSpec((32, 128), lambda i: (0, 0)),
               pl.BlockSpec((32, 128), lambda i: (0, 0))],
    out_shape=[jax.ShapeDtypeStruct((32, HID), jnp.float32),
               jax.ShapeDtypeStruct((32, 128), jnp.float32),
               jax.ShapeDtypeStruct((32, 128), jnp.float32)],
)


def kernel(x, edge_index, batch, W1, b1, ln1_w, ln1_b, W2, b2, ln2_w, ln2_b,
           Wl1, bl1, lnl1_w, lnl1_b, Wl2, bl2, lnl2_w, lnl2_b, fc_w, fc_b):
    src, dst = edge_index[0], edge_index[1]
    pad = PAD_NODE + (jnp.arange(E_PAD - N_EDGES, dtype=jnp.int32) % 40)
    srcp = jnp.concatenate([src, pad]).reshape(E_ROWS, 128)
    dstp = jnp.concatenate([dst, pad]).reshape(E_ROWS, 128)
    off2 = (jnp.arange(2, dtype=jnp.int32) * N_PAD)[:, None, None]
    off4 = (jnp.arange(4, dtype=jnp.int32) * N_PAD)[:, None, None]
    src2 = srcp[None] + off2
    src4 = srcp[None] + off4
    xp = jnp.concatenate(
        [x, jnp.zeros((N_PAD - N_NODES, IN_CH), jnp.float32)])
    batch2d = jnp.concatenate(
        [batch, jnp.full((N_PAD - N_NODES,), 99, jnp.int32)]).reshape(1, N_PAD)
    zeros128 = jnp.zeros((128, 128), jnp.float32)
    ones128 = jnp.ones((128, 128), jnp.float32)

    cnts = _sc_counts(dstp, ones128, zeros128)
    xs2 = _tc_a(xp, cnts)
    s1 = _sc_agg2(xs2.reshape(2 * N_PAD, 128), src2, dstp, zeros128)
    gmat, gs4 = _tc_b(s1, xp, cnts, W1, b1.reshape(1, HID),
                      ln1_w.reshape(1, HID), ln1_b.reshape(1, HID), W2)
    s2 = _sc_agg4(gs4.reshape(4 * N_PAD, 128), src4, dstp, zeros128)
    _, _, outd = _tc_c(s2, gmat, cnts, b2.reshape(1, HID),
                       ln2_w.reshape(1, HID), ln2_b.reshape(1, HID), batch2d,
                       jnp.asarray(_LEAGUE_L), Wl1,
                       bl1.reshape(1, HID), lnl1_w.reshape(1, HID),
                       lnl1_b.reshape(1, HID), Wl2, bl2.reshape(1, HID),
                       lnl2_w.reshape(1, HID), lnl2_b.reshape(1, HID),
                       jnp.broadcast_to(fc_w, (HID, 128)),
                       jnp.broadcast_to(fc_b.reshape(1, 1), (1, 128)))
    return outd[:NUM_TEAMS, 0]


# speedup vs baseline: 1.0132x; 1.0009x over previous
import jax
import jax.numpy as jnp
import numpy as np
from jax import lax
from jax.experimental import pallas as pl
from jax.experimental.pallas import tpu as pltpu
from jax.experimental.pallas import tpu_sc as plsc

N_NODES = 10000
N_EDGES = 160000
IN_CH = 256
HID = 512
NUM_TEAMS = 30

N_PAD = 10240
E_ROWS = 1280
E_PAD = E_ROWS * 128
PAD_NODE = 10200
STRIPE = N_PAD // 16
NB = 512
GRID = N_PAD // NB

_SC_MESH = dict(core_axis_name="c", subcore_axis_name="s")


def _league_mat() -> np.ndarray:
    iu = np.triu_indices(NUM_TEAMS, k=1)
    lsrc, ldst = iu[0], iu[1]
    deg = np.zeros((NUM_TEAMS,), np.float64)
    np.add.at(deg, ldst, 1.0)
    deg += 1.0
    dis = 1.0 / np.sqrt(deg)
    L = np.zeros((32, 32), np.float64)
    L[ldst, lsrc] = dis[lsrc] * dis[ldst]
    L[np.arange(NUM_TEAMS), np.arange(NUM_TEAMS)] = dis * dis
    return L.astype(np.float32)


_LEAGUE_L = _league_mat()


def _sc_counts_body(dst_hbm, ones_hbm, zeros_hbm, out_hbm, dstv, onesv, acc):
    c = lax.axis_index("c")
    s = lax.axis_index("s")
    w = s * 2 + c
    pltpu.sync_copy(ones_hbm, onesv)
    for z in range(5):
        pltpu.sync_copy(zeros_hbm, acc.at[pl.ds(s * STRIPE + z * 128, 128)])
    pltpu.sync_copy(dst_hbm.at[pl.ds(w * 40, 40)], dstv)
    plsc.subcore_barrier()

    def chunk(j, carry):
        pltpu.sync_copy(onesv, acc.at[dstv.at[j]], add=True)
        return carry

    lax.fori_loop(0, 40, chunk, 0)
    plsc.subcore_barrier()
    pltpu.sync_copy(acc.at[pl.ds(s * STRIPE, STRIPE)],
                    out_hbm.at[c, pl.ds(s * STRIPE, STRIPE)])


_sc_counts = pl.kernel(
    _sc_counts_body,
    out_type=jax.ShapeDtypeStruct((2, N_PAD, 128), jnp.float32),
    mesh=plsc.VectorSubcoreMesh(**_SC_MESH),
    scratch_types=[
        pltpu.VMEM((40, 128), jnp.int32),
        pltpu.VMEM((128, 128), jnp.float32),
        pltpu.VMEM_SHARED((N_PAD, 128), jnp.float32),
    ],
)


def _make_agg(n_groups):
    gpc = n_groups // 2

    def body(xs_hbm, src_hbm, dst_hbm, zeros_hbm, out_hbm,
             srcv, dstv, rowbuf, acc):
        c = lax.axis_index("c")
        s = lax.axis_index("s")
        pltpu.sync_copy(dst_hbm.at[pl.ds(s * 80, 80)], dstv)
        for it in range(gpc):
            g = it * 2 + c
            pltpu.sync_copy(src_hbm.at[g, pl.ds(s * 80, 80)], srcv)
            for z in range(5):
                pltpu.sync_copy(zeros_hbm,
                                acc.at[pl.ds(s * STRIPE + z * 128, 128)])
            plsc.subcore_barrier()

            def chunk(j, carry):
                pltpu.sync_copy(xs_hbm.at[srcv.at[j]], rowbuf)
                pltpu.sync_copy(rowbuf, acc.at[dstv.at[j]], add=True)
                return carry

            lax.fori_loop(0, 80, chunk, 0)
            plsc.subcore_barrier()
            pltpu.sync_copy(acc.at[pl.ds(s * STRIPE, STRIPE)],
                            out_hbm.at[g, pl.ds(s * STRIPE, STRIPE)])

    return pl.kernel(
        body,
        out_type=jax.ShapeDtypeStruct((n_groups, N_PAD, 128), jnp.float32),
        mesh=plsc.VectorSubcoreMesh(**_SC_MESH),
        scratch_types=[
            pltpu.VMEM((80, 128), jnp.int32),
            pltpu.VMEM((80, 128), jnp.int32),
            pltpu.VMEM((128, 128), jnp.float32),
            pltpu.VMEM_SHARED((N_PAD, 128), jnp.float32),
        ],
    )


_sc_agg2 = _make_agg(2)
_sc_agg4 = _make_agg(4)


def _dis_from_counts(cnt_ref):
    deg = cnt_ref[0, :, 0:1] + cnt_ref[1, :, 0:1] + 1.0
    dis = lax.rsqrt(deg)
    return dis


def _ln_lrelu(h, w, b, eps=1e-5):
    m = jnp.mean(h, axis=-1, keepdims=True)
    v = jnp.mean((h - m) ** 2, axis=-1, keepdims=True)
    h = (h - m) * lax.rsqrt(v + eps) * w + b
    return jnp.where(h >= 0, h, 0.01 * h)


def _tc_a_body(x_ref, cnt_ref, o_ref):
    dis = _dis_from_counts(cnt_ref)
    for g in range(2):
        o_ref[g] = dis * x_ref[:, g * 128:(g + 1) * 128]


_tc_a = pl.pallas_call(
    _tc_a_body,
    grid=(GRID,),
    in_specs=[pl.BlockSpec((NB, IN_CH), lambda i: (i, 0)),
              pl.BlockSpec((2, NB, 128), lambda i: (0, i, 0))],
    out_specs=pl.BlockSpec((2, NB, 128), lambda i: (0, i, 0)),
    out_shape=jax.ShapeDtypeStruct((2, N_PAD, 128), jnp.float32),
)


def _tc_b_body(s1_ref, x_ref, cnt_ref, w1_ref, b1_ref, lw_ref, lb_ref,
               w2_ref, go_ref, gs_ref):
    dis = _dis_from_counts(cnt_ref)
    dis2 = dis * dis
    m1 = (jnp.dot(s1_ref[0], w1_ref[0:128, :],
                  preferred_element_type=jnp.float32) +
          jnp.dot(s1_ref[1], w1_ref[128:256, :],
                  preferred_element_type=jnp.float32))
    m2 = jnp.dot(x_ref[...], w1_ref[...], preferred_element_type=jnp.float32)
    pre = dis * m1 + dis2 * m2 + b1_ref[...]
    h = _ln_lrelu(pre, lw_ref[...], lb_ref[...])
    gmat = jnp.dot(h, w2_ref[...], preferred_element_type=jnp.float32)
    go_ref[...] = gmat
    for g in range(4):
        gs_ref[g] = dis * gmat[:, g * 128:(g + 1) * 128]


_tc_b = pl.pallas_call(
    _tc_b_body,
    grid=(GRID,),
    in_specs=[pl.BlockSpec((2, NB, 128), lambda i: (0, i, 0)),
              pl.BlockSpec((NB, IN_CH), lambda i: (i, 0)),
              pl.BlockSpec((2, NB, 128), lambda i: (0, i, 0)),
              pl.BlockSpec((IN_CH, HID), lambda i: (0, 0)),
              pl.BlockSpec((1, HID), lambda i: (0, 0)),
              pl.BlockSpec((1, HID), lambda i: (0, 0)),
              pl.BlockSpec((1, HID), lambda i: (0, 0)),
              pl.BlockSpec((HID, HID), lambda i: (0, 0))],
    out_specs=[pl.BlockSpec((NB, HID), lambda i: (i, 0)),
               pl.BlockSpec((4, NB, 128), lambda i: (0, i, 0))],
    out_shape=[jax.ShapeDtypeStruct((N_PAD, HID), jnp.float32),
               jax.ShapeDtypeStruct((4, N_PAD, 128), jnp.float32)],
)


def _tc_c_body(s2_ref, g_ref, cnt_ref, b2_ref, lw_ref, lb_ref, batch_ref,
               L_ref, wl1_ref, bl1_ref, l1w_ref, l1b_ref,
               wl2_ref, bl2_ref, l2w_ref, l2b_ref, fcw_ref, fcb_ref,
               sums_ref, cnto_ref, po_ref):
    i = pl.program_id(0)
    dis = _dis_from_counts(cnt_ref)
    dis2 = dis * dis
    cat = jnp.concatenate([s2_ref[0], s2_ref[1], s2_ref[2], s2_ref[3]],
                          axis=1)
    pre = dis * cat + dis2 * g_ref[...] + b2_ref[...]
    h2 = _ln_lrelu(pre, lw_ref[...], lb_ref[...])
    ti = lax.broadcasted_iota(jnp.int32, (32, NB), 0)
    P = (batch_ref[...] == ti).astype(jnp.float32)
    su = jnp.dot(P, h2, preferred_element_type=jnp.float32)
    cn = jnp.broadcast_to(jnp.sum(P, axis=1, keepdims=True), (32, 128))

    @pl.when(i == 0)
    def _():
        sums_ref[...] = jnp.zeros_like(sums_ref)
        cnto_ref[...] = jnp.zeros_like(cnto_ref)

    sums_ref[...] += su
    cnto_ref[...] += cn

    @pl.when(i == GRID - 1)
    def _():
        cnt = jnp.maximum(cnto_ref[:, 0:1], 1.0)
        team = sums_ref[...] / cnt
        L = L_ref[...]
        tt = jnp.dot(L, jnp.dot(team, wl1_ref[...],
                                preferred_element_type=jnp.float32),
                     preferred_element_type=jnp.float32) + bl1_ref[...]
        tt = _ln_lrelu(tt, l1w_ref[...], l1b_ref[...])
        tt = jnp.dot(L, jnp.dot(tt, wl2_ref[...],
                                preferred_element_type=jnp.float32),
                     preferred_element_type=jnp.float32) + bl2_ref[...]
        tt = _ln_lrelu(tt, l2w_ref[...], l2b_ref[...])
        tt = tt + team
        po_ref[...] = jnp.dot(tt, fcw_ref[...],
                              preferred_element_type=jnp.float32) + fcb_ref[...]


_tc_c = pl.pallas_call(
    _tc_c_body,
    grid=(GRID,),
    in_specs=[pl.BlockSpec((4, NB, 128), lambda i: (0, i, 0)),
              pl.BlockSpec((NB, HID), lambda i: (i, 0)),
              pl.BlockSpec((2, NB, 128), lambda i: (0, i, 0)),
              pl.BlockSpec((1, HID), lambda i: (0, 0)),
              pl.BlockSpec((1, HID), lambda i: (0, 0)),
              pl.BlockSpec((1, HID), lambda i: (0, 0)),
              pl.BlockSpec((1, NB), lambda i: (0, i)),
              pl.BlockSpec((32, 32), lambda i: (0, 0)),
              pl.BlockSpec((HID, HID), lambda i: (0, 0)),
              pl.BlockSpec((1, HID), lambda i: (0, 0)),
              pl.BlockSpec((1, HID), lambda i: (0, 0)),
              pl.BlockSpec((1, HID), lambda i: (0, 0)),
              pl.BlockSpec((HID, HID), lambda i: (0, 0)),
              pl.BlockSpec((1, HID), lambda i: (0, 0)),
              pl.BlockSpec((1, HID), lambda i: (0, 0)),
              pl.BlockSpec((1, HID), lambda i: (0, 0)),
              pl.BlockSpec((HID, 128), lambda i: (0, 0)),
              pl.BlockSpec((1, 128), lambda i: (0, 0))],
    out_specs=[pl.BlockSpec((32, HID), lambda i: (0, 0)),
               pl.BlockSpec((32, 128), lambda i: (0, 0)),
               pl.BlockSpec((32, 128), lambda i: (0, 0))],
    out_shape=[jax.ShapeDtypeStruct((32, HID), jnp.float32),
               jax.ShapeDtypeStruct((32, 128), jnp.float32),
               jax.ShapeDtypeStruct((32, 128), jnp.float32)],
)


def kernel(x, edge_index, batch, W1, b1, ln1_w, ln1_b, W2, b2, ln2_w, ln2_b,
           Wl1, bl1, lnl1_w, lnl1_b, Wl2, bl2, lnl2_w, lnl2_b, fc_w, fc_b):
    src, dst = edge_index[0], edge_index[1]
    pad = PAD_NODE + (jnp.arange(E_PAD - N_EDGES, dtype=jnp.int32) % 40)
    srcp = jnp.concatenate([src, pad]).reshape(E_ROWS, 128)
    dstp = jnp.concatenate([dst, pad]).reshape(E_ROWS, 128)
    off2 = (jnp.arange(2, dtype=jnp.int32) * N_PAD)[:, None, None]
    off4 = (jnp.arange(4, dtype=jnp.int32) * N_PAD)[:, None, None]
    src2 = srcp[None] + off2
    src4 = srcp[None] + off4
    xp = jnp.concatenate(
        [x, jnp.zeros((N_PAD - N_NODES, IN_CH), jnp.float32)])
    batch2d = jnp.concatenate(
        [batch, jnp.full((N_PAD - N_NODES,), 99, jnp.int32)]).reshape(1, N_PAD)
    zeros128 = jnp.zeros((128, 128), jnp.float32)
    ones128 = jnp.ones((128, 128), jnp.float32)

    cnts = _sc_counts(dstp, ones128, zeros128)
    xs2 = _tc_a(xp, cnts)
    s1 = _sc_agg2(xs2.reshape(2 * N_PAD, 128), src2, dstp, zeros128)
    gmat, gs4 = _tc_b(s1, xp, cnts, W1, b1.reshape(1, HID),
                      ln1_w.reshape(1, HID), ln1_b.reshape(1, HID), W2)
    s2 = _sc_agg4(gs4.reshape(4 * N_PAD, 128), src4, dstp, zeros128)
    _, _, outd = _tc_c(s2, gmat, cnts, b2.reshape(1, HID),
                       ln2_w.reshape(1, HID), ln2_b.reshape(1, HID), batch2d,
                       jnp.asarray(_LEAGUE_L), Wl1,
                       bl1.reshape(1, HID), lnl1_w.reshape(1, HID),
                       lnl1_b.reshape(1, HID), Wl2, bl2.reshape(1, HID),
                       lnl2_w.reshape(1, HID), lnl2_b.reshape(1, HID),
                       jnp.broadcast_to(fc_w, (HID, 128)),
                       jnp.broadcast_to(fc_b.reshape(1, 1), (1, 128)))
    return outd[:NUM_TEAMS, 0]
